# baseline (speedup 1.0000x reference)
import functools

import jax
import jax.numpy as jnp
from jax import lax
from jax.experimental import pallas as pl
from jax.experimental.pallas import tpu as pltpu
from jax.experimental.pallas import tpu_sc as plsc

VOCAB = 100000
EMBED = 16
BATCH = 1024
TILE = 2048
NTILES = (VOCAB + TILE - 1) // TILE
PADV = NTILES * TILE
K = 8
CB = BATCH // K


def _gather_sc(table, idx):
    info = plsc.get_sparse_core_info()
    nc, ns = info.num_cores, info.num_subcores
    nw = nc * ns
    bpw = BATCH // nw
    mesh = plsc.VectorSubcoreMesh(core_axis_name="c", subcore_axis_name="s")

    @functools.partial(
        pl.kernel,
        mesh=mesh,
        compiler_params=pltpu.CompilerParams(use_tc_tiling_on_sc=False),
        out_type=jax.ShapeDtypeStruct((BATCH, EMBED), jnp.float32),
        scratch_types=[
            pltpu.VMEM((bpw,), jnp.int32),
            pltpu.VMEM((bpw, EMBED), jnp.float32),
            pltpu.SemaphoreType.DMA,
        ],
    )
    def gk(table_hbm, idx_hbm, out_hbm, idx_v, rows_v, sem):
        wid = lax.axis_index("s") * nc + lax.axis_index("c")
        base = wid * bpw
        pltpu.sync_copy(idx_hbm.at[pl.ds(base, bpw)], idx_v)
        pltpu.async_copy(table_hbm.at[idx_v], rows_v, sem).wait()
        pltpu.sync_copy(rows_v, out_hbm.at[pl.ds(base, bpw)])

    return gk(table, idx)


def _lse_chunk0(hb, wtp, b2p):
    def k(h_ref, w_ref, b_ref, lse_ref, m_ref, s_ref):
        j = pl.program_id(0)

        @pl.when(j == 0)
        def _():
            m_ref[...] = jnp.full((CB, 1), -1e30, jnp.float32)
            s_ref[...] = jnp.zeros((CB, 1), jnp.float32)

        logits = (
            jnp.dot(h_ref[...], w_ref[...], preferred_element_type=jnp.float32)
            + b_ref[...]
        )
        m_old = m_ref[...]
        m_new = jnp.maximum(m_old, jnp.max(logits, axis=1, keepdims=True))
        s_ref[...] = s_ref[...] * jnp.exp(m_old - m_new) + jnp.sum(
            jnp.exp(logits - m_new), axis=1, keepdims=True
        )
        m_ref[...] = m_new

        @pl.when(j == NTILES - 1)
        def _():
            lse_ref[...] = m_ref[...] + jnp.log(s_ref[...])

    return pl.pallas_call(
        k,
        grid=(NTILES,),
        in_specs=[
            pl.BlockSpec((CB, EMBED), lambda j: (0, 0)),
            pl.BlockSpec((EMBED, TILE), lambda j: (0, j)),
            pl.BlockSpec((1, TILE), lambda j: (0, j)),
        ],
        out_specs=pl.BlockSpec((CB, 1), lambda j: (0, 0)),
        out_shape=jax.ShapeDtypeStruct((CB, 1), jnp.float32),
        scratch_shapes=[
            pltpu.VMEM((CB, 1), jnp.float32),
            pltpu.VMEM((CB, 1), jnp.float32),
        ],
    )(hb[:CB], wtp, b2p)


def _main(hb, wtp, b2p, lse0):
    def k(h_ref, w_ref, b_ref, lse0_ref, o_ref, m_ref, s_ref, lse_ref):
        c = pl.program_id(0)
        j = pl.program_id(1)
        w = w_ref[:, pl.ds(pl.multiple_of(j * TILE, TILE), TILE)]
        bcol = b_ref[:, pl.ds(pl.multiple_of(j * TILE, TILE), TILE)]

        @pl.when((c == 0) & (j == 0))
        def _():
            lse_ref[pl.ds(0, CB), :] = lse0_ref[...]

        @pl.when(c < K - 1)
        def _pass1():
            r1 = (c + 1) * CB

            @pl.when(j == 0)
            def _():
                m_ref[...] = jnp.full((CB, 1), -1e30, jnp.float32)
                s_ref[...] = jnp.zeros((CB, 1), jnp.float32)

            hc = h_ref[pl.ds(r1, CB), :]
            logits = jnp.dot(hc, w, preferred_element_type=jnp.float32) + bcol
            m_old = m_ref[...]
            m_new = jnp.maximum(m_old, jnp.max(logits, axis=1, keepdims=True))
            s_ref[...] = s_ref[...] * jnp.exp(m_old - m_new) + jnp.sum(
                jnp.exp(logits - m_new), axis=1, keepdims=True
            )
            m_ref[...] = m_new

            @pl.when(j == NTILES - 1)
            def _():
                lse_ref[pl.ds(r1, CB), :] = m_ref[...] + jnp.log(s_ref[...])

        r0 = c * CB
        hc0 = h_ref[pl.ds(r0, CB), :]
        logits0 = jnp.dot(hc0, w, preferred_element_type=jnp.float32)
        o_ref[...] = logits0 + (bcol - lse_ref[pl.ds(r0, CB), :])

    return pl.pallas_call(
        k,
        grid=(K, NTILES),
        in_specs=[
            pl.BlockSpec((BATCH, EMBED), lambda c, j: (0, 0)),
            pl.BlockSpec((EMBED, PADV), lambda c, j: (0, 0)),
            pl.BlockSpec((1, PADV), lambda c, j: (0, 0)),
            pl.BlockSpec((CB, 1), lambda c, j: (0, 0)),
        ],
        out_specs=pl.BlockSpec((CB, TILE), lambda c, j: (c, j)),
        out_shape=jax.ShapeDtypeStruct((BATCH, VOCAB), jnp.float32),
        scratch_shapes=[
            pltpu.VMEM((CB, 1), jnp.float32),
            pltpu.VMEM((CB, 1), jnp.float32),
            pltpu.VMEM((BATCH, 1), jnp.float32),
        ],
    )(hb, wtp, b2p, lse0)


def kernel(x, table, W, b):
    h = _gather_sc(table, x.astype(jnp.int32))
    hb = h.astype(jnp.bfloat16)
    wtp = jnp.concatenate(
        [W.T, jnp.zeros((EMBED, PADV - VOCAB), jnp.float32)], axis=1
    ).astype(jnp.bfloat16)
    b2p = jnp.concatenate(
        [b.reshape(1, VOCAB), jnp.full((1, PADV - VOCAB), -1e30, jnp.float32)],
        axis=1,
    )
    lse0 = _lse_chunk0(hb, wtp, b2p)
    return _main(hb, wtp, b2p, lse0)


# speedup vs baseline: 1.0042x; 1.0042x over previous
import functools

import jax
import jax.numpy as jnp
from jax import lax
from jax.experimental import pallas as pl
from jax.experimental.pallas import tpu as pltpu
from jax.experimental.pallas import tpu_sc as plsc

VOCAB = 100000
EMBED = 16
BATCH = 1024
TILE = 2048
NTILES = (VOCAB + TILE - 1) // TILE
PADV = NTILES * TILE
LASTW = VOCAB - (NTILES - 1) * TILE
K = 8
CB = BATCH // K
NQ = 4


def _gather_sc(table, idx):
    info = plsc.get_sparse_core_info()
    nc, ns = info.num_cores, info.num_subcores
    nw = nc * ns
    bpw = BATCH // nw
    mesh = plsc.VectorSubcoreMesh(core_axis_name="c", subcore_axis_name="s")

    @functools.partial(
        pl.kernel,
        mesh=mesh,
        compiler_params=pltpu.CompilerParams(use_tc_tiling_on_sc=False),
        out_type=jax.ShapeDtypeStruct((BATCH, EMBED), jnp.float32),
        scratch_types=[
            pltpu.VMEM((bpw,), jnp.int32),
            pltpu.VMEM((bpw, EMBED), jnp.float32),
            pltpu.SemaphoreType.DMA,
        ],
    )
    def gk(table_hbm, idx_hbm, out_hbm, idx_v, rows_v, sem):
        wid = lax.axis_index("s") * nc + lax.axis_index("c")
        base = wid * bpw
        pltpu.sync_copy(idx_hbm.at[pl.ds(base, bpw)], idx_v)
        pltpu.async_copy(table_hbm.at[idx_v], rows_v, sem).wait()
        pltpu.sync_copy(rows_v, out_hbm.at[pl.ds(base, bpw)])

    return gk(table, idx)


def _fused(h, wtp, b2p):
    T = (K + 1) * NTILES

    def full_copy(o_hbm, obuf, sems, q, c, j):
        return pltpu.make_async_copy(
            obuf.at[q],
            o_hbm.at[pl.ds(c * CB, CB), pl.ds(j * TILE, TILE)],
            sems.at[q],
        )

    def tail_copy(o_hbm, tbuf, sems, q, c):
        return pltpu.make_async_copy(
            tbuf.at[q],
            o_hbm.at[pl.ds(c * CB, CB), pl.ds((NTILES - 1) * TILE, LASTW)],
            sems.at[q],
        )

    def k(h_ref, w_ref, b_ref, o_hbm, obuf, tbuf, sems, m_ref, s_ref, lse_ref):
        t = pl.program_id(0)
        p = t // NTILES
        j = lax.rem(t, NTILES)
        woff = pl.multiple_of(j * TILE, TILE)
        w = w_ref[:, pl.ds(woff, TILE)]
        bcol = b_ref[:, pl.ds(woff, TILE)]

        @pl.when(p < K)
        def _pass1():
            r1 = p * CB

            @pl.when(j == 0)
            def _():
                m_ref[...] = jnp.full((CB, 1), -1e30, jnp.float32)
                s_ref[...] = jnp.zeros((CB, 1), jnp.float32)

            hc = h_ref[pl.ds(r1, CB), :].astype(jnp.bfloat16)
            logits = jnp.dot(hc, w, preferred_element_type=jnp.float32) + bcol
            m_old = m_ref[...]
            m_new = jnp.maximum(m_old, jnp.max(logits, axis=1, keepdims=True))
            s_ref[...] = s_ref[...] * jnp.exp(m_old - m_new) + jnp.sum(
                jnp.exp(logits - m_new), axis=1, keepdims=True
            )
            m_ref[...] = m_new

            @pl.when(j == NTILES - 1)
            def _():
                lse_ref[pl.ds(r1, CB), :] = m_ref[...] + jnp.log(s_ref[...])

        @pl.when(p >= 1)
        def _pass2():
            i = t - NTILES
            q = lax.rem(i, NQ)
            r0 = (p - 1) * CB

            @pl.when(i >= NQ)
            def _wait_prev():
                cp = jnp.where(j >= NQ, p - 1, p - 2)
                jp = jnp.where(j >= NQ, j - NQ, j + NTILES - NQ)

                @pl.when(j != NQ - 1)
                def _():
                    full_copy(o_hbm, obuf, sems, q, cp, jp).wait()

                @pl.when(j == NQ - 1)
                def _():
                    tail_copy(o_hbm, tbuf, sems, q, cp).wait()

            hc = h_ref[pl.ds(r0, CB), :].astype(jnp.bfloat16)
            logits = jnp.dot(hc, w, preferred_element_type=jnp.float32)
            vals = logits + (bcol - lse_ref[pl.ds(r0, CB), :])

            @pl.when(j != NTILES - 1)
            def _():
                obuf[pl.ds(q, 1)] = vals[None]
                full_copy(o_hbm, obuf, sems, q, p - 1, j).start()

            @pl.when(j == NTILES - 1)
            def _():
                tbuf[pl.ds(q, 1)] = vals[:, :LASTW][None]
                tail_copy(o_hbm, tbuf, sems, q, p - 1).start()

            @pl.when(t == T - 1)
            def _drain():
                for d in range(NQ):
                    jd = NTILES - NQ + d
                    qd = lax.rem(K * NTILES - NQ + d, NQ)
                    if jd == NTILES - 1:
                        tail_copy(o_hbm, tbuf, sems, qd, K - 1).wait()
                    else:
                        full_copy(o_hbm, obuf, sems, qd, K - 1, jd).wait()

    return pl.pallas_call(
        k,
        grid=(T,),
        in_specs=[
            pl.BlockSpec((BATCH, EMBED), lambda t: (0, 0)),
            pl.BlockSpec((EMBED, PADV), lambda t: (0, 0)),
            pl.BlockSpec((1, PADV), lambda t: (0, 0)),
        ],
        out_specs=pl.BlockSpec(memory_space=pl.ANY),
        out_shape=jax.ShapeDtypeStruct((BATCH, VOCAB), jnp.float32),
        scratch_shapes=[
            pltpu.VMEM((NQ, CB, TILE), jnp.float32),
            pltpu.VMEM((NQ, CB, LASTW), jnp.float32),
            pltpu.SemaphoreType.DMA((NQ,)),
            pltpu.VMEM((CB, 1), jnp.float32),
            pltpu.VMEM((CB, 1), jnp.float32),
            pltpu.VMEM((BATCH, 1), jnp.float32),
        ],
    )(h, wtp, b2p)


def kernel(x, table, W, b):
    h = _gather_sc(table, x.astype(jnp.int32))
    wtp = jnp.concatenate(
        [W.T, jnp.zeros((EMBED, PADV - VOCAB), jnp.float32)], axis=1
    ).astype(jnp.bfloat16)
    b2p = jnp.concatenate(
        [b.reshape(1, VOCAB), jnp.full((1, PADV - VOCAB), -1e30, jnp.float32)],
        axis=1,
    )
    return _fused(h, wtp, b2p)
